# baseline (speedup 1.0000x reference)
import functools

import jax
import jax.numpy as jnp
from jax.experimental import pallas as pl
from jax.experimental.pallas import tpu as pltpu

_B = 4
_NC = 1024
_NF = 4096
_M = 4096
_TN = 256
_NT = (_NC + _NF) // _TN
_NCT = _NC // _TN
_EPS = 1e-12


def _cham_body(x_ref, yt_ref, sums_ref, colmin_scr):
    i = pl.program_id(1)

    @pl.when(i == 0)
    def _init():
        sums_ref[0, 0, 0] = 0.0
        sums_ref[0, 0, 1] = 0.0
        sums_ref[0, 0, 2] = 0.0
        sums_ref[0, 0, 3] = 0.0
        colmin_scr[...] = jnp.full((2, _M), jnp.inf, jnp.float32)

    x = x_ref[0]
    x0 = x[:, 0:1]
    x1 = x[:, 1:2]
    x2 = x[:, 2:3]
    y = yt_ref[0]
    y0 = y[0:1, :]
    y1 = y[1:2, :]
    y2 = y[2:3, :]

    xs = x0 * x0 + x1 * x1 + x2 * x2
    ys = y0 * y0 + y1 * y1 + y2 * y2
    x0b = x0.astype(jnp.bfloat16).astype(jnp.float32)
    x1b = x1.astype(jnp.bfloat16).astype(jnp.float32)
    x2b = x2.astype(jnp.bfloat16).astype(jnp.float32)
    y0b = y0.astype(jnp.bfloat16).astype(jnp.float32)
    y1b = y1.astype(jnp.bfloat16).astype(jnp.float32)
    y2b = y2.astype(jnp.bfloat16).astype(jnp.float32)
    xy = x0b * y0b + x1b * y1b + x2b * y2b
    d = (xs + ys) - 2.0 * xy
    d = jnp.maximum(d, 0.0)

    rowmin = jnp.min(d, axis=1, keepdims=True)
    colmin = jnp.min(d, axis=0, keepdims=True)
    s = jnp.sum(jnp.sqrt(rowmin + _EPS))

    is_c = i < _NCT
    sums_ref[0, 0, 0] += jnp.where(is_c, s, 0.0)
    sums_ref[0, 0, 1] += jnp.where(is_c, 0.0, s)

    cm0 = colmin_scr[0:1, :]
    cm1 = colmin_scr[1:2, :]
    colmin_scr[0:1, :] = jnp.where(is_c, jnp.minimum(cm0, colmin), cm0)
    colmin_scr[1:2, :] = jnp.where(is_c, cm1, jnp.minimum(cm1, colmin))

    @pl.when(i == _NT - 1)
    def _fin():
        sums_ref[0, 0, 2] = jnp.sum(jnp.sqrt(colmin_scr[0:1, :] + _EPS))
        sums_ref[0, 0, 3] = jnp.sum(jnp.sqrt(colmin_scr[1:2, :] + _EPS))


@functools.partial(jax.jit, static_argnames=())
def kernel(coarse, fine, gt, alpha):
    x_all = jnp.concatenate([coarse, fine], axis=1)
    yt = jnp.transpose(gt, (0, 2, 1))

    sums = pl.pallas_call(
        _cham_body,
        grid=(_B, _NT),
        in_specs=[
            pl.BlockSpec((1, _TN, 3), lambda b, i: (b, i, 0)),
            pl.BlockSpec((1, 3, _M), lambda b, i: (b, 0, 0)),
        ],
        out_specs=pl.BlockSpec((1, 1, 4), lambda b, i: (b, 0, 0),
                               memory_space=pltpu.SMEM),
        out_shape=jax.ShapeDtypeStruct((_B, 1, 4), jnp.float32),
        scratch_shapes=[pltpu.VMEM((2, _M), jnp.float32)],
    )(x_all, yt)

    tot = jnp.sum(sums[:, 0, :], axis=0)
    mean_c2g = tot[0] / (_B * _NC)
    mean_f2g = tot[1] / (_B * _NF)
    mean_g2c = tot[2] / (_B * _M)
    mean_g2f = tot[3] / (_B * _M)
    dcd_c = mean_g2c + 0.1 * mean_c2g
    dcd_f = mean_g2f + 0.1 * mean_f2g
    return dcd_c + alpha * dcd_f


# speedup vs baseline: 1.3152x; 1.3152x over previous
import functools

import jax
import jax.numpy as jnp
from jax.experimental import pallas as pl
from jax.experimental.pallas import tpu as pltpu

_B = 4
_NC = 1024
_NF = 4096
_M = 4096
_TN = 256
_NT = (_NC + _NF) // _TN
_NCT = _NC // _TN
_EPS = 1e-12
_NEG = -1e30


def _cham_body(x_ref, y_ref, sums_ref, colmax_scr):
    i = pl.program_id(1)

    @pl.when(i == 0)
    def _init():
        sums_ref[0, 0, 0] = 0.0
        sums_ref[0, 0, 1] = 0.0
        colmax_scr[...] = jnp.full((2, _M), _NEG, jnp.float32)

    e = jax.lax.dot_general(
        x_ref[0], y_ref[0],
        dimension_numbers=(((1,), (0,)), ((), ())),
        preferred_element_type=jnp.float32,
    )

    rowmax = jnp.max(e, axis=1, keepdims=True)
    dmin = jnp.maximum(-2.0 * rowmax, 0.0)
    s = jnp.sum(jnp.sqrt(dmin + _EPS))

    is_c = i < _NCT
    sums_ref[0, 0, 0] += jnp.where(is_c, s, 0.0)
    sums_ref[0, 0, 1] += jnp.where(is_c, 0.0, s)

    colmax = jnp.max(e, axis=0, keepdims=True)
    cm0 = colmax_scr[0:1, :]
    cm1 = colmax_scr[1:2, :]
    colmax_scr[0:1, :] = jnp.where(is_c, jnp.maximum(cm0, colmax), cm0)
    colmax_scr[1:2, :] = jnp.where(is_c, cm1, jnp.maximum(cm1, colmax))

    @pl.when(i == _NT - 1)
    def _fin():
        d0 = jnp.maximum(-2.0 * colmax_scr[0:1, :], 0.0)
        d1 = jnp.maximum(-2.0 * colmax_scr[1:2, :], 0.0)
        sums_ref[0, 0, 2] = jnp.sum(jnp.sqrt(d0 + _EPS))
        sums_ref[0, 0, 3] = jnp.sum(jnp.sqrt(d1 + _EPS))


def _augment(pts, left):
    cb = pts.astype(jnp.bfloat16)
    a = -0.5 * jnp.sum(pts * pts, axis=-1, keepdims=True)
    a_hi = a.astype(jnp.bfloat16)
    a_lo = (a - a_hi.astype(jnp.float32)).astype(jnp.bfloat16)
    one = jnp.ones_like(a_hi)
    zero = jnp.zeros_like(a_hi)
    if left:
        cols = [cb, a_hi, a_lo, one, one, zero]
    else:
        cols = [cb, one, one, a_hi, a_lo, zero]
    return jnp.concatenate(cols, axis=-1)


@functools.partial(jax.jit, static_argnames=())
def kernel(coarse, fine, gt, alpha):
    x_all = jnp.concatenate([coarse, fine], axis=1)
    xa = _augment(x_all, left=True)
    ya = jnp.transpose(_augment(gt, left=False), (0, 2, 1))

    sums = pl.pallas_call(
        _cham_body,
        grid=(_B, _NT),
        in_specs=[
            pl.BlockSpec((1, _TN, 8), lambda b, i: (b, i, 0)),
            pl.BlockSpec((1, 8, _M), lambda b, i: (b, 0, 0)),
        ],
        out_specs=pl.BlockSpec((1, 1, 4), lambda b, i: (b, 0, 0),
                               memory_space=pltpu.SMEM),
        out_shape=jax.ShapeDtypeStruct((_B, 1, 4), jnp.float32),
        scratch_shapes=[pltpu.VMEM((2, _M), jnp.float32)],
    )(xa, ya)

    tot = jnp.sum(sums[:, 0, :], axis=0)
    mean_c2g = tot[0] / (_B * _NC)
    mean_f2g = tot[1] / (_B * _NF)
    mean_g2c = tot[2] / (_B * _M)
    mean_g2f = tot[3] / (_B * _M)
    dcd_c = mean_g2c + 0.1 * mean_c2g
    dcd_f = mean_g2f + 0.1 * mean_f2g
    return dcd_c + alpha * dcd_f
